# baseline (speedup 1.0000x reference)
import functools

import jax
import jax.numpy as jnp
from jax import lax
from jax.experimental import pallas as pl
from jax.experimental.pallas import tpu as pltpu
from jax.experimental.pallas import tpu_sc as plsc

N = 10000
E = 320000
D = 128
NP = 10240
NW = 32
EPW = E // NW
S = 80
NSUB = 5
C = S * NSUB
NCH = EPW // C
RPS = NP // 16

_mesh = plsc.VectorSubcoreMesh(core_axis_name="c", subcore_axis_name="s")


def _mmT(a, w):
    return lax.dot_general(a, w, (((1,), (1,)), ((), ())),
                           preferred_element_type=jnp.float32)


RPW = EPW // S


def _al(x, n):
    return pl.multiple_of(x, n)


@functools.partial(
    pl.kernel,
    out_type=jax.ShapeDtypeStruct((2, NP), jnp.float32),
    mesh=_mesh,
    scratch_types=[
        pltpu.VMEM((RPW, S), jnp.int32),
        pltpu.VMEM((S,), jnp.float32),
        pltpu.VMEM_SHARED((NP,), jnp.float32),
    ],
)
def _deg_kernel(row3d, zeros_n, ones_s, degp, ridx, ones_v, deg_sh):
    c = lax.axis_index("c")
    s = lax.axis_index("s")
    wid = c * 16 + s
    pltpu.sync_copy(ones_s, ones_v)
    pltpu.sync_copy(zeros_n.at[pl.ds(_al(s * RPS, RPS), RPS)],
                    deg_sh.at[pl.ds(_al(s * RPS, RPS), RPS)])
    pltpu.sync_copy(row3d.at[wid], ridx)
    plsc.subcore_barrier()

    def body(r, carry):
        pltpu.sync_copy(ones_v, deg_sh.at[ridx.at[r]], add=True)
        return carry

    lax.fori_loop(0, RPW, body, 0)
    plsc.subcore_barrier()
    pltpu.sync_copy(deg_sh.at[pl.ds(_al(s * RPS, RPS), RPS)],
                    degp.at[c, pl.ds(_al(s * RPS, RPS), RPS)])


@functools.partial(
    pl.kernel,
    out_type=jax.ShapeDtypeStruct((E,), jnp.float32),
    mesh=_mesh,
    scratch_types=[
        pltpu.VMEM((RPW, S), jnp.int32),
        pltpu.VMEM((RPW, S), jnp.int32),
        pltpu.VMEM((C,), jnp.float32),
        pltpu.VMEM((C,), jnp.float32),
        pltpu.VMEM((C,), jnp.float32),
        pltpu.SemaphoreType.DMA,
    ],
)
def _norm_kernel(row3d, col3d, dis, norm, ridx, cidx, dr_v, dc_v, nbuf, sem):
    c = lax.axis_index("c")
    s = lax.axis_index("s")
    wid = c * 16 + s
    pltpu.sync_copy(row3d.at[wid], ridx)
    pltpu.sync_copy(col3d.at[wid], cidx)

    def chunk(i, carry):
        descs = []
        for j in range(NSUB):
            rr = i * NSUB + j
            descs.append(pltpu.async_copy(
                dis.at[ridx.at[rr]], dr_v.at[pl.ds(j * S, S)], sem))
            descs.append(pltpu.async_copy(
                dis.at[cidx.at[rr]], dc_v.at[pl.ds(j * S, S)], sem))
        for d in descs:
            d.wait()

        def grp(g, cc):
            nbuf[pl.ds(g * 16, 16)] = (dr_v[pl.ds(g * 16, 16)]
                                       * dc_v[pl.ds(g * 16, 16)])
            return cc

        lax.fori_loop(0, C // 16, grp, 0)
        pltpu.sync_copy(nbuf, norm.at[pl.ds(_al(wid * EPW + i * C, 8), C)])
        return carry

    lax.fori_loop(0, NCH, chunk, 0)


DH = D // 2
EPS = E // 16
CE = 160
NSE = CE // S
NCHE = EPS // CE


@functools.partial(
    pl.kernel,
    out_type=jax.ShapeDtypeStruct((2, NP, DH), jnp.float32),
    mesh=_mesh,
    scratch_types=[
        pltpu.VMEM((3, NSE, S), jnp.int32),
        pltpu.VMEM((3, NSE, S), jnp.int32),
        pltpu.VMEM((3, CE), jnp.float32),
        pltpu.VMEM((2, CE, DH), jnp.float32),
        pltpu.VMEM((2, CE, DH), jnp.float32),
        pltpu.VMEM_SHARED((NP, DH), jnp.float32),
        pltpu.SemaphoreType.DMA((3,)),
        pltpu.SemaphoreType.DMA((2,)),
        pltpu.SemaphoreType.DMA((2,)),
    ],
    compiler_params=pltpu.CompilerParams(use_tc_tiling_on_sc=False),
)
def _edge_kernel(hxs, eembs, row4d, col4d, norm, zeros_big, aggrp,
                 ridx, cidx, nrm, gath, eembv, aggr_sh, semA, semG, semS):
    c = lax.axis_index("c")
    s = lax.axis_index("s")
    pltpu.sync_copy(zeros_big.at[pl.ds(_al(s * RPS, RPS), RPS)],
                    aggr_sh.at[pl.ds(_al(s * RPS, RPS), RPS)])
    plsc.subcore_barrier()

    def issue_a(i, a):
        e0 = _al(s * EPS + i * CE, 8)
        pltpu.async_copy(row4d.at[s, i], ridx.at[a], semA.at[a])
        pltpu.async_copy(col4d.at[s, i], cidx.at[a], semA.at[a])
        pltpu.async_copy(norm.at[pl.ds(e0, CE)], nrm.at[a], semA.at[a])

    def wait_a(a):
        pltpu.make_async_copy(row4d.at[s, 0], ridx.at[a], semA.at[a]).wait()
        pltpu.make_async_copy(col4d.at[s, 0], cidx.at[a], semA.at[a]).wait()
        pltpu.make_async_copy(norm.at[pl.ds(0, CE)], nrm.at[a],
                              semA.at[a]).wait()

    def issue_g(i, a, b):
        e0 = _al(s * EPS + i * CE, 8)
        for j in range(NSE):
            pltpu.async_copy(hxs.at[c].at[ridx.at[a, j]],
                             gath.at[b, pl.ds(j * S, S)], semG.at[b])
        pltpu.async_copy(eembs.at[c, pl.ds(e0, CE)], eembv.at[b],
                         semG.at[b])

    def wait_g(b):
        for j in range(NSE):
            pltpu.make_async_copy(eembs.at[c, pl.ds(0, S)],
                                  gath.at[b, pl.ds(j * S, S)],
                                  semG.at[b]).wait()
        pltpu.make_async_copy(eembs.at[c, pl.ds(0, CE)], eembv.at[b],
                              semG.at[b]).wait()

    def issue_s(a, b):
        for j in range(NSE):
            pltpu.async_copy(gath.at[b, pl.ds(j * S, S)],
                             aggr_sh.at[cidx.at[a, j]], semS.at[b],
                             add=True)

    def wait_s(b):
        for j in range(NSE):
            pltpu.make_async_copy(eembs.at[c, pl.ds(0, S)],
                                  gath.at[b, pl.ds(j * S, S)],
                                  semS.at[b]).wait()

    def compute(a, b):
        def grpfn(g, cc):
            nv = nrm[a, pl.ds(g * 16, 16)]
            for ee in range(16):
                r = g * 16 + ee
                sc_n = nv[ee]
                for f in range(DH // 16):
                    gv = gath[b, r, pl.ds(f * 16, 16)]
                    em = eembv[b, r, pl.ds(f * 16, 16)]
                    gath[b, r, pl.ds(f * 16, 16)] = (
                        jnp.maximum(gv + em, 0.0) * sc_n)
            return cc

        lax.fori_loop(0, CE // 16, grpfn, 0)

    issue_a(0, 0)
    issue_a(1, 1)
    wait_a(0)
    issue_g(0, 0, 0)

    def chunk(i, carry):
        a = lax.rem(i, 3)
        an = lax.rem(i + 1, 3)
        an2 = lax.rem(i + 2, 3)
        b = lax.rem(i, 2)
        bn = lax.rem(i + 1, 2)

        @pl.when(i >= 1)
        def _():
            wait_s(bn)

        @pl.when(i < NCHE - 1)
        def _():
            wait_a(an)
            issue_g(i + 1, an, bn)

        @pl.when(i < NCHE - 2)
        def _():
            issue_a(i + 2, an2)

        wait_g(b)
        compute(a, b)
        issue_s(a, b)
        return carry

    lax.fori_loop(0, NCHE, chunk, 0)
    wait_s((NCHE - 1) % 2)
    plsc.subcore_barrier()
    pltpu.sync_copy(aggr_sh.at[pl.ds(_al(s * RPS, RPS), RPS)],
                    aggrp.at[c, pl.ds(_al(s * RPS, RPS), RPS)])


def _enc_body(x_ref, wn_ref, bn_ref, w0_ref, b0_ref, out_ref):
    h0 = _mmT(x_ref[...], wn_ref[...]) + bn_ref[...]
    out_ref[...] = _mmT(h0, w0_ref[...]) + b0_ref[...]


def _encode(x_pad, wn, bn, w0, b0):
    return pl.pallas_call(
        _enc_body,
        grid=(NP // 1024,),
        in_specs=[
            pl.BlockSpec((1024, D), lambda i: (i, 0)),
            pl.BlockSpec((D, D), lambda i: (0, 0)),
            pl.BlockSpec((1, D), lambda i: (0, 0)),
            pl.BlockSpec((D, D), lambda i: (0, 0)),
            pl.BlockSpec((1, D), lambda i: (0, 0)),
        ],
        out_specs=pl.BlockSpec((1024, D), lambda i: (i, 0)),
        out_shape=jax.ShapeDtypeStruct((NP, D), jnp.float32),
    )(x_pad, wn, bn, w0, b0)


def _eemb_body(ea_ref, we_ref, be_ref, out_ref):
    out_ref[...] = (_mmT(ea_ref[...], we_ref[...]) + be_ref[0])[None]


def _eemb(edge_attr, we, be2):
    return pl.pallas_call(
        _eemb_body,
        grid=(2, E // 4000),
        in_specs=[
            pl.BlockSpec((4000, 16), lambda c, i: (i, 0)),
            pl.BlockSpec((DH, 16), lambda c, i: (c, 0)),
            pl.BlockSpec((1, 1, DH), lambda c, i: (c, 0, 0)),
        ],
        out_specs=pl.BlockSpec((1, 4000, DH), lambda c, i: (c, i, 0)),
        out_shape=jax.ShapeDtypeStruct((2, E, DH), jnp.float32),
    )(edge_attr, we, be2)


def _degfin_body(degp_ref, dis_ref, invbc_ref):
    d = degp_ref[0] + degp_ref[1] + 1.0
    dis_ref[...] = lax.rsqrt(d)
    inv = 1.0 / d
    r0 = lax.broadcasted_iota(jnp.int32, (1024, 8), 0)
    c0 = lax.broadcasted_iota(jnp.int32, (1024, 8), 1)
    p = jnp.where(r0 // 128 == c0, 1.0, 0.0)
    r1 = lax.broadcasted_iota(jnp.int32, (1024, D), 0)
    c1 = lax.broadcasted_iota(jnp.int32, (1024, D), 1)
    m = jnp.where(r1 % D == c1, 1.0, 0.0)
    b1 = lax.dot_general(p, inv, (((1,), (0,)), ((), ())),
                         preferred_element_type=jnp.float32)
    invbc_ref[...] = lax.dot_general(
        b1 * m, jnp.ones((D, D), jnp.float32), (((1,), (0,)), ((), ())),
        preferred_element_type=jnp.float32)


def _degfin(degp2d):
    return pl.pallas_call(
        _degfin_body,
        grid=(NP // 1024,),
        in_specs=[pl.BlockSpec((2, 8, D), lambda i: (0, i, 0))],
        out_specs=[
            pl.BlockSpec((8, D), lambda i: (i, 0)),
            pl.BlockSpec((1024, D), lambda i: (i, 0)),
        ],
        out_shape=[
            jax.ShapeDtypeStruct((NP // D, D), jnp.float32),
            jax.ShapeDtypeStruct((NP, D), jnp.float32),
        ],
    )(degp2d)


def _g1_body(a0_ref, a1_ref, hx_ref, invbc_ref, root_ref,
             t_ref, sum_ref, sumsq_ref):
    i = pl.program_id(0)
    aggr = jnp.concatenate([a0_ref[0], a1_ref[0]], axis=1)
    t = (aggr
         + jnp.maximum(hx_ref[...] + root_ref[...], 0.0) * invbc_ref[...])
    rowid = lax.broadcasted_iota(jnp.int32, (1024, D), 0) + i * 1024
    t = jnp.where(rowid < N, t, 0.0)
    t_ref[...] = t
    ps = jnp.sum(t, axis=0, keepdims=True)
    pss = jnp.sum(t * t, axis=0, keepdims=True)

    @pl.when(i == 0)
    def _():
        sum_ref[...] = ps
        sumsq_ref[...] = pss

    @pl.when(i > 0)
    def _():
        sum_ref[...] += ps
        sumsq_ref[...] += pss


def _g1(aggrp, hx, invbc, root_l):
    return pl.pallas_call(
        _g1_body,
        grid=(NP // 1024,),
        in_specs=[
            pl.BlockSpec((1, 1024, DH), lambda i: (0, i, 0)),
            pl.BlockSpec((1, 1024, DH), lambda i: (1, i, 0)),
            pl.BlockSpec((1024, D), lambda i: (i, 0)),
            pl.BlockSpec((1024, D), lambda i: (i, 0)),
            pl.BlockSpec((1, D), lambda i: (0, 0)),
        ],
        out_specs=[
            pl.BlockSpec((1024, D), lambda i: (i, 0)),
            pl.BlockSpec((1, D), lambda i: (0, 0)),
            pl.BlockSpec((1, D), lambda i: (0, 0)),
        ],
        out_shape=[
            jax.ShapeDtypeStruct((NP, D), jnp.float32),
            jax.ShapeDtypeStruct((1, D), jnp.float32),
            jax.ShapeDtypeStruct((1, D), jnp.float32),
        ],
    )(aggrp, aggrp, hx, invbc, root_l)


def _bn_apply(t_ref, sum_ref, sumsq_ref, g_ref, b_ref):
    mean = sum_ref[...] * (1.0 / N)
    var = sumsq_ref[...] * (1.0 / N) - mean * mean
    scale = g_ref[...] * lax.rsqrt(var + 1e-5)
    return (t_ref[...] - mean) * scale + b_ref[...]


def _g2mid_body(t_ref, sum_ref, sumsq_ref, g_ref, b_ref, w_ref, bl_ref,
                out_ref):
    h = jnp.maximum(_bn_apply(t_ref, sum_ref, sumsq_ref, g_ref, b_ref), 0.0)
    out_ref[...] = _mmT(h, w_ref[...]) + bl_ref[...]


def _g2mid(t, sums, sumsq, gamma, beta, w_next, b_next):
    return pl.pallas_call(
        _g2mid_body,
        grid=(NP // 1024,),
        in_specs=[
            pl.BlockSpec((1024, D), lambda i: (i, 0)),
            pl.BlockSpec((1, D), lambda i: (0, 0)),
            pl.BlockSpec((1, D), lambda i: (0, 0)),
            pl.BlockSpec((1, D), lambda i: (0, 0)),
            pl.BlockSpec((1, D), lambda i: (0, 0)),
            pl.BlockSpec((D, D), lambda i: (0, 0)),
            pl.BlockSpec((1, D), lambda i: (0, 0)),
        ],
        out_specs=pl.BlockSpec((1024, D), lambda i: (i, 0)),
        out_shape=jax.ShapeDtypeStruct((NP, D), jnp.float32),
    )(t, sums, sumsq, gamma, beta, w_next, b_next)


def _g2last_body(t_ref, sum_ref, sumsq_ref, g_ref, b_ref, out_ref):
    out_ref[...] = _bn_apply(t_ref, sum_ref, sumsq_ref, g_ref, b_ref)


def _g2last(t, sums, sumsq, gamma, beta):
    return pl.pallas_call(
        _g2last_body,
        grid=(NP // 1024,),
        in_specs=[
            pl.BlockSpec((1024, D), lambda i: (i, 0)),
            pl.BlockSpec((1, D), lambda i: (0, 0)),
            pl.BlockSpec((1, D), lambda i: (0, 0)),
            pl.BlockSpec((1, D), lambda i: (0, 0)),
            pl.BlockSpec((1, D), lambda i: (0, 0)),
        ],
        out_specs=pl.BlockSpec((1024, D), lambda i: (i, 0)),
        out_shape=jax.ShapeDtypeStruct((NP, D), jnp.float32),
    )(t, sums, sumsq, gamma, beta)


def kernel(x, edge_index, edge_attr, batch, W_node, b_node, W_lin, b_lin,
           root_emb, W_edge, b_edge, bn_gamma, bn_beta):
    del batch
    row3d = edge_index[0].astype(jnp.int32).reshape(NW, RPW, S)
    col3d = edge_index[1].astype(jnp.int32).reshape(NW, RPW, S)
    x_pad = jnp.concatenate(
        [x, jnp.zeros((NP - N, D), jnp.float32)], axis=0)
    zeros_n = jnp.zeros((NP,), jnp.float32)
    ones_s = jnp.ones((S,), jnp.float32)
    zeros_big = jnp.zeros((NP, DH), jnp.float32)
    row4d = edge_index[0].astype(jnp.int32).reshape(16, NCHE, NSE, S)
    col4d = edge_index[1].astype(jnp.int32).reshape(16, NCHE, NSE, S)

    degp = _deg_kernel(row3d, zeros_n, ones_s)
    dis2d, invbc = _degfin(degp.reshape(2, NP // D, D))
    norm = _norm_kernel(row3d, col3d, dis2d.reshape(NP))

    hx = _encode(x_pad, W_node, b_node.reshape(1, D),
                 W_lin[0], b_lin[0].reshape(1, D))
    h = None
    for l in range(4):
        eembs = _eemb(edge_attr, W_edge[l], b_edge[l].reshape(2, 1, DH))
        hxs = hx.reshape(NP, 2, DH).transpose(1, 0, 2)
        aggrp = _edge_kernel(hxs, eembs, row4d, col4d, norm, zeros_big)
        t, sums, sumsq = _g1(aggrp, hx, invbc, root_emb[l].reshape(1, D))
        if l < 3:
            hx = _g2mid(t, sums, sumsq, bn_gamma[l].reshape(1, D),
                        bn_beta[l].reshape(1, D),
                        W_lin[l + 1], b_lin[l + 1].reshape(1, D))
        else:
            h = _g2last(t, sums, sumsq, bn_gamma[l].reshape(1, D),
                        bn_beta[l].reshape(1, D))
    return h[:N]


# speedup vs baseline: 1.1874x; 1.1874x over previous
import functools

import jax
import jax.numpy as jnp
from jax import lax
from jax.experimental import pallas as pl
from jax.experimental.pallas import tpu as pltpu
from jax.experimental.pallas import tpu_sc as plsc

N = 10000
E = 320000
D = 128
NP = 10240
NW = 32
EPW = E // NW
S = 80
NSUB = 5
C = S * NSUB
NCH = EPW // C
RPS = NP // 16

_mesh = plsc.VectorSubcoreMesh(core_axis_name="c", subcore_axis_name="s")


def _mmT(a, w):
    return lax.dot_general(a, w, (((1,), (1,)), ((), ())),
                           preferred_element_type=jnp.float32)


RPW = EPW // S


def _al(x, n):
    return pl.multiple_of(x, n)


@functools.partial(
    pl.kernel,
    out_type=jax.ShapeDtypeStruct((2, NP), jnp.float32),
    mesh=_mesh,
    scratch_types=[
        pltpu.VMEM((RPW, S), jnp.int32),
        pltpu.VMEM((S,), jnp.float32),
        pltpu.VMEM_SHARED((NP,), jnp.float32),
    ],
)
def _deg_kernel(row3d, zeros_n, ones_s, degp, ridx, ones_v, deg_sh):
    c = lax.axis_index("c")
    s = lax.axis_index("s")
    wid = c * 16 + s
    pltpu.sync_copy(ones_s, ones_v)
    pltpu.sync_copy(zeros_n.at[pl.ds(_al(s * RPS, RPS), RPS)],
                    deg_sh.at[pl.ds(_al(s * RPS, RPS), RPS)])
    pltpu.sync_copy(row3d.at[wid], ridx)
    plsc.subcore_barrier()

    def body(r, carry):
        pltpu.sync_copy(ones_v, deg_sh.at[ridx.at[r]], add=True)
        return carry

    lax.fori_loop(0, RPW, body, 0)
    plsc.subcore_barrier()
    pltpu.sync_copy(deg_sh.at[pl.ds(_al(s * RPS, RPS), RPS)],
                    degp.at[c, pl.ds(_al(s * RPS, RPS), RPS)])


@functools.partial(
    pl.kernel,
    out_type=jax.ShapeDtypeStruct((E,), jnp.float32),
    mesh=_mesh,
    scratch_types=[
        pltpu.VMEM((RPW, S), jnp.int32),
        pltpu.VMEM((RPW, S), jnp.int32),
        pltpu.VMEM((C,), jnp.float32),
        pltpu.VMEM((C,), jnp.float32),
        pltpu.VMEM((C,), jnp.float32),
        pltpu.SemaphoreType.DMA,
    ],
)
def _norm_kernel(row3d, col3d, dis, norm, ridx, cidx, dr_v, dc_v, nbuf, sem):
    c = lax.axis_index("c")
    s = lax.axis_index("s")
    wid = c * 16 + s
    pltpu.sync_copy(row3d.at[wid], ridx)
    pltpu.sync_copy(col3d.at[wid], cidx)

    def chunk(i, carry):
        descs = []
        for j in range(NSUB):
            rr = i * NSUB + j
            descs.append(pltpu.async_copy(
                dis.at[ridx.at[rr]], dr_v.at[pl.ds(j * S, S)], sem))
            descs.append(pltpu.async_copy(
                dis.at[cidx.at[rr]], dc_v.at[pl.ds(j * S, S)], sem))
        for d in descs:
            d.wait()

        def grp(g, cc):
            nbuf[pl.ds(g * 16, 16)] = (dr_v[pl.ds(g * 16, 16)]
                                       * dc_v[pl.ds(g * 16, 16)])
            return cc

        lax.fori_loop(0, C // 16, grp, 0)
        pltpu.sync_copy(nbuf, norm.at[pl.ds(_al(wid * EPW + i * C, 8), C)])
        return carry

    lax.fori_loop(0, NCH, chunk, 0)


DH = D // 2
EPS = E // 16
CE = 160
NSE = CE // S
NCHE = EPS // CE


@functools.partial(
    pl.kernel,
    out_type=jax.ShapeDtypeStruct((2, NP, DH), jnp.float32),
    mesh=_mesh,
    scratch_types=[
        pltpu.VMEM((4, NSE, S), jnp.int32),
        pltpu.VMEM((4, NSE, S), jnp.int32),
        pltpu.VMEM((4, CE), jnp.float32),
        pltpu.VMEM((2, CE, DH), jnp.float32),
        pltpu.VMEM((2, CE, DH), jnp.float32),
        pltpu.VMEM_SHARED((NP, DH), jnp.float32),
        pltpu.SemaphoreType.DMA((4,)),
        pltpu.SemaphoreType.DMA((2,)),
        pltpu.SemaphoreType.DMA((2,)),
    ],
    compiler_params=pltpu.CompilerParams(use_tc_tiling_on_sc=False),
)
def _edge_kernel(hxs, eembs, row4d, col4d, norm, zeros_big, aggrp,
                 ridx, cidx, nrm, gath, eembv, aggr_sh, semA, semG, semS):
    c = lax.axis_index("c")
    s = lax.axis_index("s")
    pltpu.sync_copy(zeros_big.at[pl.ds(_al(s * RPS, RPS), RPS)],
                    aggr_sh.at[pl.ds(_al(s * RPS, RPS), RPS)])
    plsc.subcore_barrier()

    def issue_a(i, a):
        e0 = _al(s * EPS + i * CE, 8)
        pltpu.async_copy(row4d.at[s, i], ridx.at[a], semA.at[a])
        pltpu.async_copy(col4d.at[s, i], cidx.at[a], semA.at[a])
        pltpu.async_copy(norm.at[pl.ds(e0, CE)], nrm.at[a], semA.at[a])

    def wait_a(a):
        pltpu.make_async_copy(row4d.at[s, 0], ridx.at[a], semA.at[a]).wait()
        pltpu.make_async_copy(col4d.at[s, 0], cidx.at[a], semA.at[a]).wait()
        pltpu.make_async_copy(norm.at[pl.ds(0, CE)], nrm.at[a],
                              semA.at[a]).wait()

    def issue_g(i, a, b):
        e0 = _al(s * EPS + i * CE, 8)
        for j in range(NSE):
            pltpu.async_copy(hxs.at[c].at[ridx.at[a, j]],
                             gath.at[b, pl.ds(j * S, S)], semG.at[b])
        pltpu.async_copy(eembs.at[c, pl.ds(e0, CE)], eembv.at[b],
                         semG.at[b])

    def wait_g(b):
        for j in range(NSE):
            pltpu.make_async_copy(eembs.at[c, pl.ds(0, S)],
                                  gath.at[b, pl.ds(j * S, S)],
                                  semG.at[b]).wait()
        pltpu.make_async_copy(eembs.at[c, pl.ds(0, CE)], eembv.at[b],
                              semG.at[b]).wait()

    def issue_s(a, b):
        for j in range(NSE):
            pltpu.async_copy(gath.at[b, pl.ds(j * S, S)],
                             aggr_sh.at[cidx.at[a, j]], semS.at[b],
                             add=True)

    def wait_s(b):
        for j in range(NSE):
            pltpu.make_async_copy(eembs.at[c, pl.ds(0, S)],
                                  gath.at[b, pl.ds(j * S, S)],
                                  semS.at[b]).wait()

    def compute(a, b):
        def grpfn(g, cc):
            nv = nrm[a, pl.ds(g * 16, 16)]
            for ee in range(16):
                r = g * 16 + ee
                sc_n = nv[ee]
                for f in range(DH // 16):
                    gv = gath[b, r, pl.ds(f * 16, 16)]
                    em = eembv[b, r, pl.ds(f * 16, 16)]
                    gath[b, r, pl.ds(f * 16, 16)] = (
                        jnp.maximum(gv + em, 0.0) * sc_n)
            return cc

        lax.fori_loop(0, CE // 16, grpfn, 0)

    issue_a(0, 0)
    issue_a(1, 1)
    wait_a(0)
    issue_g(0, 0, 0)

    def quad(io, carry):
        i0 = io * 4
        for k in range(4):
            i = i0 + k
            a, an, an2 = k, (k + 1) % 4, (k + 2) % 4
            b, bn = k % 2, (k + 1) % 2

            @pl.when(i >= 1)
            def _():
                wait_s(bn)

            wait_a(an)
            issue_g(i + 1, an, bn)

            @pl.when(i + 2 < NCHE)
            def _():
                issue_a(i + 2, an2)

            wait_g(b)
            compute(a, b)
            issue_s(a, b)
        return carry

    lax.fori_loop(0, (NCHE - 1) // 4, quad, 0)
    wait_s(1)
    wait_g(0)
    compute(0, 0)
    issue_s(0, 0)
    wait_s(0)
    plsc.subcore_barrier()
    pltpu.sync_copy(aggr_sh.at[pl.ds(_al(s * RPS, RPS), RPS)],
                    aggrp.at[c, pl.ds(_al(s * RPS, RPS), RPS)])


def _enc_body(x_ref, wn_ref, bn_ref, w0_ref, b0_ref, out_ref):
    h0 = _mmT(x_ref[...], wn_ref[...]) + bn_ref[...]
    out_ref[...] = _mmT(h0, w0_ref[...]) + b0_ref[...]


def _encode(x_pad, wn, bn, w0, b0):
    return pl.pallas_call(
        _enc_body,
        grid=(NP // 1024,),
        in_specs=[
            pl.BlockSpec((1024, D), lambda i: (i, 0)),
            pl.BlockSpec((D, D), lambda i: (0, 0)),
            pl.BlockSpec((1, D), lambda i: (0, 0)),
            pl.BlockSpec((D, D), lambda i: (0, 0)),
            pl.BlockSpec((1, D), lambda i: (0, 0)),
        ],
        out_specs=pl.BlockSpec((1024, D), lambda i: (i, 0)),
        out_shape=jax.ShapeDtypeStruct((NP, D), jnp.float32),
    )(x_pad, wn, bn, w0, b0)


def _eemb_body(ea_ref, we_ref, be_ref, out_ref):
    out_ref[...] = (_mmT(ea_ref[...], we_ref[...]) + be_ref[0])[None]


def _eemb(edge_attr, we, be2):
    return pl.pallas_call(
        _eemb_body,
        grid=(2, E // 4000),
        in_specs=[
            pl.BlockSpec((4000, 16), lambda c, i: (i, 0)),
            pl.BlockSpec((DH, 16), lambda c, i: (c, 0)),
            pl.BlockSpec((1, 1, DH), lambda c, i: (c, 0, 0)),
        ],
        out_specs=pl.BlockSpec((1, 4000, DH), lambda c, i: (c, i, 0)),
        out_shape=jax.ShapeDtypeStruct((2, E, DH), jnp.float32),
    )(edge_attr, we, be2)


def _degfin_body(degp_ref, dis_ref, invbc_ref):
    d = degp_ref[0] + degp_ref[1] + 1.0
    dis_ref[...] = lax.rsqrt(d)
    inv = 1.0 / d
    r0 = lax.broadcasted_iota(jnp.int32, (1024, 8), 0)
    c0 = lax.broadcasted_iota(jnp.int32, (1024, 8), 1)
    p = jnp.where(r0 // 128 == c0, 1.0, 0.0)
    r1 = lax.broadcasted_iota(jnp.int32, (1024, D), 0)
    c1 = lax.broadcasted_iota(jnp.int32, (1024, D), 1)
    m = jnp.where(r1 % D == c1, 1.0, 0.0)
    b1 = lax.dot_general(p, inv, (((1,), (0,)), ((), ())),
                         preferred_element_type=jnp.float32)
    invbc_ref[...] = lax.dot_general(
        b1 * m, jnp.ones((D, D), jnp.float32), (((1,), (0,)), ((), ())),
        preferred_element_type=jnp.float32)


def _degfin(degp2d):
    return pl.pallas_call(
        _degfin_body,
        grid=(NP // 1024,),
        in_specs=[pl.BlockSpec((2, 8, D), lambda i: (0, i, 0))],
        out_specs=[
            pl.BlockSpec((8, D), lambda i: (i, 0)),
            pl.BlockSpec((1024, D), lambda i: (i, 0)),
        ],
        out_shape=[
            jax.ShapeDtypeStruct((NP // D, D), jnp.float32),
            jax.ShapeDtypeStruct((NP, D), jnp.float32),
        ],
    )(degp2d)


def _g1_body(a0_ref, a1_ref, hx_ref, invbc_ref, root_ref,
             t_ref, sum_ref, sumsq_ref):
    i = pl.program_id(0)
    aggr = jnp.concatenate([a0_ref[0], a1_ref[0]], axis=1)
    t = (aggr
         + jnp.maximum(hx_ref[...] + root_ref[...], 0.0) * invbc_ref[...])
    rowid = lax.broadcasted_iota(jnp.int32, (1024, D), 0) + i * 1024
    t = jnp.where(rowid < N, t, 0.0)
    t_ref[...] = t
    ps = jnp.sum(t, axis=0, keepdims=True)
    pss = jnp.sum(t * t, axis=0, keepdims=True)

    @pl.when(i == 0)
    def _():
        sum_ref[...] = ps
        sumsq_ref[...] = pss

    @pl.when(i > 0)
    def _():
        sum_ref[...] += ps
        sumsq_ref[...] += pss


def _g1(aggrp, hx, invbc, root_l):
    return pl.pallas_call(
        _g1_body,
        grid=(NP // 1024,),
        in_specs=[
            pl.BlockSpec((1, 1024, DH), lambda i: (0, i, 0)),
            pl.BlockSpec((1, 1024, DH), lambda i: (1, i, 0)),
            pl.BlockSpec((1024, D), lambda i: (i, 0)),
            pl.BlockSpec((1024, D), lambda i: (i, 0)),
            pl.BlockSpec((1, D), lambda i: (0, 0)),
        ],
        out_specs=[
            pl.BlockSpec((1024, D), lambda i: (i, 0)),
            pl.BlockSpec((1, D), lambda i: (0, 0)),
            pl.BlockSpec((1, D), lambda i: (0, 0)),
        ],
        out_shape=[
            jax.ShapeDtypeStruct((NP, D), jnp.float32),
            jax.ShapeDtypeStruct((1, D), jnp.float32),
            jax.ShapeDtypeStruct((1, D), jnp.float32),
        ],
    )(aggrp, aggrp, hx, invbc, root_l)


def _bn_apply(t_ref, sum_ref, sumsq_ref, g_ref, b_ref):
    mean = sum_ref[...] * (1.0 / N)
    var = sumsq_ref[...] * (1.0 / N) - mean * mean
    scale = g_ref[...] * lax.rsqrt(var + 1e-5)
    return (t_ref[...] - mean) * scale + b_ref[...]


def _g2mid_body(t_ref, sum_ref, sumsq_ref, g_ref, b_ref, w_ref, bl_ref,
                out_ref):
    h = jnp.maximum(_bn_apply(t_ref, sum_ref, sumsq_ref, g_ref, b_ref), 0.0)
    out_ref[...] = _mmT(h, w_ref[...]) + bl_ref[...]


def _g2mid(t, sums, sumsq, gamma, beta, w_next, b_next):
    return pl.pallas_call(
        _g2mid_body,
        grid=(NP // 1024,),
        in_specs=[
            pl.BlockSpec((1024, D), lambda i: (i, 0)),
            pl.BlockSpec((1, D), lambda i: (0, 0)),
            pl.BlockSpec((1, D), lambda i: (0, 0)),
            pl.BlockSpec((1, D), lambda i: (0, 0)),
            pl.BlockSpec((1, D), lambda i: (0, 0)),
            pl.BlockSpec((D, D), lambda i: (0, 0)),
            pl.BlockSpec((1, D), lambda i: (0, 0)),
        ],
        out_specs=pl.BlockSpec((1024, D), lambda i: (i, 0)),
        out_shape=jax.ShapeDtypeStruct((NP, D), jnp.float32),
    )(t, sums, sumsq, gamma, beta, w_next, b_next)


def _g2last_body(t_ref, sum_ref, sumsq_ref, g_ref, b_ref, out_ref):
    out_ref[...] = _bn_apply(t_ref, sum_ref, sumsq_ref, g_ref, b_ref)


def _g2last(t, sums, sumsq, gamma, beta):
    return pl.pallas_call(
        _g2last_body,
        grid=(NP // 1024,),
        in_specs=[
            pl.BlockSpec((1024, D), lambda i: (i, 0)),
            pl.BlockSpec((1, D), lambda i: (0, 0)),
            pl.BlockSpec((1, D), lambda i: (0, 0)),
            pl.BlockSpec((1, D), lambda i: (0, 0)),
            pl.BlockSpec((1, D), lambda i: (0, 0)),
        ],
        out_specs=pl.BlockSpec((1024, D), lambda i: (i, 0)),
        out_shape=jax.ShapeDtypeStruct((NP, D), jnp.float32),
    )(t, sums, sumsq, gamma, beta)


def kernel(x, edge_index, edge_attr, batch, W_node, b_node, W_lin, b_lin,
           root_emb, W_edge, b_edge, bn_gamma, bn_beta):
    del batch
    row3d = edge_index[0].astype(jnp.int32).reshape(NW, RPW, S)
    col3d = edge_index[1].astype(jnp.int32).reshape(NW, RPW, S)
    x_pad = jnp.concatenate(
        [x, jnp.zeros((NP - N, D), jnp.float32)], axis=0)
    zeros_n = jnp.zeros((NP,), jnp.float32)
    ones_s = jnp.ones((S,), jnp.float32)
    zeros_big = jnp.zeros((NP, DH), jnp.float32)
    row4d = edge_index[0].astype(jnp.int32).reshape(16, NCHE, NSE, S)
    col4d = edge_index[1].astype(jnp.int32).reshape(16, NCHE, NSE, S)

    degp = _deg_kernel(row3d, zeros_n, ones_s)
    dis2d, invbc = _degfin(degp.reshape(2, NP // D, D))
    norm = _norm_kernel(row3d, col3d, dis2d.reshape(NP))

    hx = _encode(x_pad, W_node, b_node.reshape(1, D),
                 W_lin[0], b_lin[0].reshape(1, D))
    h = None
    for l in range(4):
        eembs = _eemb(edge_attr, W_edge[l], b_edge[l].reshape(2, 1, DH))
        hxs = hx.reshape(NP, 2, DH).transpose(1, 0, 2)
        aggrp = _edge_kernel(hxs, eembs, row4d, col4d, norm, zeros_big)
        t, sums, sumsq = _g1(aggrp, hx, invbc, root_emb[l].reshape(1, D))
        if l < 3:
            hx = _g2mid(t, sums, sumsq, bn_gamma[l].reshape(1, D),
                        bn_beta[l].reshape(1, D),
                        W_lin[l + 1], b_lin[l + 1].reshape(1, D))
        else:
            h = _g2last(t, sums, sumsq, bn_gamma[l].reshape(1, D),
                        bn_beta[l].reshape(1, D))
    return h[:N]


# speedup vs baseline: 1.4661x; 1.2347x over previous
import functools

import jax
import jax.numpy as jnp
from jax import lax
from jax.experimental import pallas as pl
from jax.experimental.pallas import tpu as pltpu
from jax.experimental.pallas import tpu_sc as plsc

N = 10000
E = 320000
D = 128
NP = 10240
NW = 32
EPW = E // NW
S = 80
NSUB = 5
C = S * NSUB
NCH = EPW // C
RPS = NP // 16

_mesh = plsc.VectorSubcoreMesh(core_axis_name="c", subcore_axis_name="s")


def _mmT(a, w):
    return lax.dot_general(a, w, (((1,), (1,)), ((), ())),
                           preferred_element_type=jnp.float32)


RPW = EPW // S


def _al(x, n):
    return pl.multiple_of(x, n)


@functools.partial(
    pl.kernel,
    out_type=jax.ShapeDtypeStruct((2, NP), jnp.float32),
    mesh=_mesh,
    scratch_types=[
        pltpu.VMEM((RPW, S), jnp.int32),
        pltpu.VMEM((S,), jnp.float32),
        pltpu.VMEM_SHARED((NP,), jnp.float32),
    ],
)
def _deg_kernel(row3d, zeros_n, ones_s, degp, ridx, ones_v, deg_sh):
    c = lax.axis_index("c")
    s = lax.axis_index("s")
    wid = c * 16 + s
    pltpu.sync_copy(ones_s, ones_v)
    pltpu.sync_copy(zeros_n.at[pl.ds(_al(s * RPS, RPS), RPS)],
                    deg_sh.at[pl.ds(_al(s * RPS, RPS), RPS)])
    pltpu.sync_copy(row3d.at[wid], ridx)
    plsc.subcore_barrier()

    def body(r, carry):
        pltpu.sync_copy(ones_v, deg_sh.at[ridx.at[r]], add=True)
        return carry

    lax.fori_loop(0, RPW, body, 0)
    plsc.subcore_barrier()
    pltpu.sync_copy(deg_sh.at[pl.ds(_al(s * RPS, RPS), RPS)],
                    degp.at[c, pl.ds(_al(s * RPS, RPS), RPS)])


@functools.partial(
    pl.kernel,
    out_type=jax.ShapeDtypeStruct((E,), jnp.float32),
    mesh=_mesh,
    scratch_types=[
        pltpu.VMEM((RPW, S), jnp.int32),
        pltpu.VMEM((RPW, S), jnp.int32),
        pltpu.VMEM((C,), jnp.float32),
        pltpu.VMEM((C,), jnp.float32),
        pltpu.VMEM((C,), jnp.float32),
        pltpu.SemaphoreType.DMA,
    ],
)
def _norm_kernel(row3d, col3d, dis, norm, ridx, cidx, dr_v, dc_v, nbuf, sem):
    c = lax.axis_index("c")
    s = lax.axis_index("s")
    wid = c * 16 + s
    pltpu.sync_copy(row3d.at[wid], ridx)
    pltpu.sync_copy(col3d.at[wid], cidx)

    def chunk(i, carry):
        descs = []
        for j in range(NSUB):
            rr = i * NSUB + j
            descs.append(pltpu.async_copy(
                dis.at[ridx.at[rr]], dr_v.at[pl.ds(j * S, S)], sem))
            descs.append(pltpu.async_copy(
                dis.at[cidx.at[rr]], dc_v.at[pl.ds(j * S, S)], sem))
        for d in descs:
            d.wait()

        def grp(g, cc):
            nbuf[pl.ds(g * 16, 16)] = (dr_v[pl.ds(g * 16, 16)]
                                       * dc_v[pl.ds(g * 16, 16)])
            return cc

        lax.fori_loop(0, C // 16, grp, 0)
        pltpu.sync_copy(nbuf, norm.at[pl.ds(_al(wid * EPW + i * C, 8), C)])
        return carry

    lax.fori_loop(0, NCH, chunk, 0)


DH = D // 2
EPS = E // 16
CE = 160
NSE = CE // S
NCHE = EPS // CE


@functools.partial(
    pl.kernel,
    out_type=jax.ShapeDtypeStruct((2, NP, DH), jnp.float32),
    mesh=_mesh,
    scratch_types=[
        pltpu.VMEM((4, NSE, S), jnp.int32),
        pltpu.VMEM((4, NSE, S), jnp.int32),
        pltpu.VMEM((4, CE), jnp.float32),
        pltpu.VMEM((2, CE, DH), jnp.float32),
        pltpu.VMEM((2, CE // 2, D), jnp.float32),
        pltpu.VMEM_SHARED((NP, DH), jnp.float32),
        pltpu.SemaphoreType.DMA((4,)),
        pltpu.SemaphoreType.DMA((2,)),
        pltpu.SemaphoreType.DMA((2,)),
    ],
    compiler_params=pltpu.CompilerParams(use_tc_tiling_on_sc=False),
)
def _edge_kernel(hxs, eembs, row4d, col4d, norm, zeros_big, aggrp,
                 ridx, cidx, nrm, gath, eembv, aggr_sh, semA, semG, semS):
    c = lax.axis_index("c")
    s = lax.axis_index("s")
    pltpu.sync_copy(zeros_big.at[pl.ds(_al(s * RPS, RPS), RPS)],
                    aggr_sh.at[pl.ds(_al(s * RPS, RPS), RPS)])
    plsc.subcore_barrier()

    def issue_a(i, a):
        e0 = _al(s * EPS + i * CE, 8)
        pltpu.async_copy(row4d.at[s, i], ridx.at[a], semA.at[a])
        pltpu.async_copy(col4d.at[s, i], cidx.at[a], semA.at[a])
        pltpu.async_copy(norm.at[pl.ds(e0, CE)], nrm.at[a], semA.at[a])

    def wait_a(a):
        pltpu.make_async_copy(row4d.at[s, 0], ridx.at[a], semA.at[a]).wait()
        pltpu.make_async_copy(col4d.at[s, 0], cidx.at[a], semA.at[a]).wait()
        pltpu.make_async_copy(norm.at[pl.ds(0, CE)], nrm.at[a],
                              semA.at[a]).wait()

    def issue_g(i, a, b):
        e0h = _al(s * (EPS // 2) + i * (CE // 2), 8)
        for j in range(NSE):
            pltpu.async_copy(hxs.at[c].at[ridx.at[a, j]],
                             gath.at[b, pl.ds(j * S, S)], semG.at[b])
        pltpu.async_copy(eembs.at[c, pl.ds(e0h, CE // 2)], eembv.at[b],
                         semG.at[b])

    def wait_g(b):
        for j in range(NSE):
            pltpu.make_async_copy(hxs.at[c, pl.ds(0, S)],
                                  gath.at[b, pl.ds(j * S, S)],
                                  semG.at[b]).wait()
        pltpu.make_async_copy(eembs.at[c, pl.ds(0, CE // 2)], eembv.at[b],
                              semG.at[b]).wait()

    def issue_s(a, b):
        for j in range(NSE):
            pltpu.async_copy(gath.at[b, pl.ds(j * S, S)],
                             aggr_sh.at[cidx.at[a, j]], semS.at[b],
                             add=True)

    def wait_s(b):
        for j in range(NSE):
            pltpu.make_async_copy(hxs.at[c, pl.ds(0, S)],
                                  gath.at[b, pl.ds(j * S, S)],
                                  semS.at[b]).wait()

    def compute(a, b):
        def grpfn(g, cc):
            nv = nrm[a, pl.ds(g * 16, 16)]
            for ee in range(16):
                r = g * 16 + ee
                sc_n = nv[ee]
                for f in range(DH // 16):
                    gv = gath[b, r, pl.ds(f * 16, 16)]
                    em = eembv[b, g * 8 + ee // 2,
                               pl.ds((ee % 2) * DH + f * 16, 16)]
                    gath[b, r, pl.ds(f * 16, 16)] = (
                        jnp.maximum(gv + em, 0.0) * sc_n)
            return cc

        lax.fori_loop(0, CE // 16, grpfn, 0)

    issue_a(0, 0)
    issue_a(1, 1)
    wait_a(0)
    issue_g(0, 0, 0)

    def quad(io, carry):
        i0 = io * 4
        for k in range(4):
            i = i0 + k
            a, an, an2 = k, (k + 1) % 4, (k + 2) % 4
            b, bn = k % 2, (k + 1) % 2

            @pl.when(i >= 1)
            def _():
                wait_s(bn)

            wait_a(an)
            issue_g(i + 1, an, bn)

            @pl.when(i + 2 < NCHE)
            def _():
                issue_a(i + 2, an2)

            wait_g(b)
            compute(a, b)
            issue_s(a, b)
        return carry

    lax.fori_loop(0, (NCHE - 1) // 4, quad, 0)
    wait_s(1)
    wait_g(0)
    compute(0, 0)
    issue_s(0, 0)
    wait_s(0)
    plsc.subcore_barrier()
    pltpu.sync_copy(aggr_sh.at[pl.ds(_al(s * RPS, RPS), RPS)],
                    aggrp.at[c, pl.ds(_al(s * RPS, RPS), RPS)])


def _enc_body(x_ref, wn_ref, bn_ref, w0_ref, b0_ref, out_ref):
    h0 = _mmT(x_ref[...], wn_ref[...]) + bn_ref[...]
    out_ref[...] = _mmT(h0, w0_ref[...]) + b0_ref[...]


def _encode(x_pad, wn, bn, w0, b0):
    return pl.pallas_call(
        _enc_body,
        grid=(NP // 1024,),
        in_specs=[
            pl.BlockSpec((1024, D), lambda i: (i, 0)),
            pl.BlockSpec((D, D), lambda i: (0, 0)),
            pl.BlockSpec((1, D), lambda i: (0, 0)),
            pl.BlockSpec((D, D), lambda i: (0, 0)),
            pl.BlockSpec((1, D), lambda i: (0, 0)),
        ],
        out_specs=pl.BlockSpec((1024, D), lambda i: (i, 0)),
        out_shape=jax.ShapeDtypeStruct((NP, D), jnp.float32),
    )(x_pad, wn, bn, w0, b0)


def _eemb_body(eae_ref, eao_ref, we_ref, be_ref, out_ref):
    ev = _mmT(eae_ref[...], we_ref[...]) + be_ref[0]
    od = _mmT(eao_ref[...], we_ref[...]) + be_ref[0]
    out_ref[...] = jnp.concatenate([ev, od], axis=1)[None]


def _eemb(ea_even, ea_odd, we, be2):
    return pl.pallas_call(
        _eemb_body,
        grid=(2, (E // 2) // 2000),
        in_specs=[
            pl.BlockSpec((2000, 16), lambda c, i: (i, 0)),
            pl.BlockSpec((2000, 16), lambda c, i: (i, 0)),
            pl.BlockSpec((DH, 16), lambda c, i: (c, 0)),
            pl.BlockSpec((1, 1, DH), lambda c, i: (c, 0, 0)),
        ],
        out_specs=pl.BlockSpec((1, 2000, D), lambda c, i: (c, i, 0)),
        out_shape=jax.ShapeDtypeStruct((2, E // 2, D), jnp.float32),
    )(ea_even, ea_odd, we, be2)


def _degfin_body(degp_ref, dis_ref, invbc_ref):
    d = degp_ref[0] + degp_ref[1] + 1.0
    dis_ref[...] = lax.rsqrt(d)
    inv = 1.0 / d
    r0 = lax.broadcasted_iota(jnp.int32, (1024, 8), 0)
    c0 = lax.broadcasted_iota(jnp.int32, (1024, 8), 1)
    p = jnp.where(r0 // 128 == c0, 1.0, 0.0)
    r1 = lax.broadcasted_iota(jnp.int32, (1024, D), 0)
    c1 = lax.broadcasted_iota(jnp.int32, (1024, D), 1)
    m = jnp.where(r1 % D == c1, 1.0, 0.0)
    b1 = lax.dot_general(p, inv, (((1,), (0,)), ((), ())),
                         preferred_element_type=jnp.float32)
    invbc_ref[...] = lax.dot_general(
        b1 * m, jnp.ones((D, D), jnp.float32), (((1,), (0,)), ((), ())),
        preferred_element_type=jnp.float32)


def _degfin(degp2d):
    return pl.pallas_call(
        _degfin_body,
        grid=(NP // 1024,),
        in_specs=[pl.BlockSpec((2, 8, D), lambda i: (0, i, 0))],
        out_specs=[
            pl.BlockSpec((8, D), lambda i: (i, 0)),
            pl.BlockSpec((1024, D), lambda i: (i, 0)),
        ],
        out_shape=[
            jax.ShapeDtypeStruct((NP // D, D), jnp.float32),
            jax.ShapeDtypeStruct((NP, D), jnp.float32),
        ],
    )(degp2d)


def _g1_body(a0_ref, a1_ref, hx_ref, invbc_ref, root_ref,
             t_ref, sum_ref, sumsq_ref):
    i = pl.program_id(0)
    aggr = jnp.concatenate([a0_ref[0], a1_ref[0]], axis=1)
    t = (aggr
         + jnp.maximum(hx_ref[...] + root_ref[...], 0.0) * invbc_ref[...])
    rowid = lax.broadcasted_iota(jnp.int32, (1024, D), 0) + i * 1024
    t = jnp.where(rowid < N, t, 0.0)
    t_ref[...] = t
    ps = jnp.sum(t, axis=0, keepdims=True)
    pss = jnp.sum(t * t, axis=0, keepdims=True)

    @pl.when(i == 0)
    def _():
        sum_ref[...] = ps
        sumsq_ref[...] = pss

    @pl.when(i > 0)
    def _():
        sum_ref[...] += ps
        sumsq_ref[...] += pss


def _g1(aggrp, hx, invbc, root_l):
    return pl.pallas_call(
        _g1_body,
        grid=(NP // 1024,),
        in_specs=[
            pl.BlockSpec((1, 1024, DH), lambda i: (0, i, 0)),
            pl.BlockSpec((1, 1024, DH), lambda i: (1, i, 0)),
            pl.BlockSpec((1024, D), lambda i: (i, 0)),
            pl.BlockSpec((1024, D), lambda i: (i, 0)),
            pl.BlockSpec((1, D), lambda i: (0, 0)),
        ],
        out_specs=[
            pl.BlockSpec((1024, D), lambda i: (i, 0)),
            pl.BlockSpec((1, D), lambda i: (0, 0)),
            pl.BlockSpec((1, D), lambda i: (0, 0)),
        ],
        out_shape=[
            jax.ShapeDtypeStruct((NP, D), jnp.float32),
            jax.ShapeDtypeStruct((1, D), jnp.float32),
            jax.ShapeDtypeStruct((1, D), jnp.float32),
        ],
    )(aggrp, aggrp, hx, invbc, root_l)


def _bn_apply(t_ref, sum_ref, sumsq_ref, g_ref, b_ref):
    mean = sum_ref[...] * (1.0 / N)
    var = sumsq_ref[...] * (1.0 / N) - mean * mean
    scale = g_ref[...] * lax.rsqrt(var + 1e-5)
    return (t_ref[...] - mean) * scale + b_ref[...]


def _g2mid_body(t_ref, sum_ref, sumsq_ref, g_ref, b_ref, w_ref, bl_ref,
                out_ref):
    h = jnp.maximum(_bn_apply(t_ref, sum_ref, sumsq_ref, g_ref, b_ref), 0.0)
    out_ref[...] = _mmT(h, w_ref[...]) + bl_ref[...]


def _g2mid(t, sums, sumsq, gamma, beta, w_next, b_next):
    return pl.pallas_call(
        _g2mid_body,
        grid=(NP // 1024,),
        in_specs=[
            pl.BlockSpec((1024, D), lambda i: (i, 0)),
            pl.BlockSpec((1, D), lambda i: (0, 0)),
            pl.BlockSpec((1, D), lambda i: (0, 0)),
            pl.BlockSpec((1, D), lambda i: (0, 0)),
            pl.BlockSpec((1, D), lambda i: (0, 0)),
            pl.BlockSpec((D, D), lambda i: (0, 0)),
            pl.BlockSpec((1, D), lambda i: (0, 0)),
        ],
        out_specs=pl.BlockSpec((1024, D), lambda i: (i, 0)),
        out_shape=jax.ShapeDtypeStruct((NP, D), jnp.float32),
    )(t, sums, sumsq, gamma, beta, w_next, b_next)


def _g2last_body(t_ref, sum_ref, sumsq_ref, g_ref, b_ref, out_ref):
    out_ref[...] = _bn_apply(t_ref, sum_ref, sumsq_ref, g_ref, b_ref)


def _g2last(t, sums, sumsq, gamma, beta):
    return pl.pallas_call(
        _g2last_body,
        grid=(NP // 1024,),
        in_specs=[
            pl.BlockSpec((1024, D), lambda i: (i, 0)),
            pl.BlockSpec((1, D), lambda i: (0, 0)),
            pl.BlockSpec((1, D), lambda i: (0, 0)),
            pl.BlockSpec((1, D), lambda i: (0, 0)),
            pl.BlockSpec((1, D), lambda i: (0, 0)),
        ],
        out_specs=pl.BlockSpec((1024, D), lambda i: (i, 0)),
        out_shape=jax.ShapeDtypeStruct((NP, D), jnp.float32),
    )(t, sums, sumsq, gamma, beta)


def kernel(x, edge_index, edge_attr, batch, W_node, b_node, W_lin, b_lin,
           root_emb, W_edge, b_edge, bn_gamma, bn_beta):
    del batch
    row3d = edge_index[0].astype(jnp.int32).reshape(NW, RPW, S)
    col3d = edge_index[1].astype(jnp.int32).reshape(NW, RPW, S)
    x_pad = jnp.concatenate(
        [x, jnp.zeros((NP - N, D), jnp.float32)], axis=0)
    zeros_n = jnp.zeros((NP,), jnp.float32)
    ones_s = jnp.ones((S,), jnp.float32)
    zeros_big = jnp.zeros((NP, DH), jnp.float32)
    row4d = edge_index[0].astype(jnp.int32).reshape(16, NCHE, NSE, S)
    col4d = edge_index[1].astype(jnp.int32).reshape(16, NCHE, NSE, S)

    degp = _deg_kernel(row3d, zeros_n, ones_s)
    dis2d, invbc = _degfin(degp.reshape(2, NP // D, D))
    norm = _norm_kernel(row3d, col3d, dis2d.reshape(NP))

    hx = _encode(x_pad, W_node, b_node.reshape(1, D),
                 W_lin[0], b_lin[0].reshape(1, D))
    ea_even = edge_attr[0::2]
    ea_odd = edge_attr[1::2]
    h = None
    for l in range(4):
        eembs = _eemb(ea_even, ea_odd, W_edge[l], b_edge[l].reshape(2, 1, DH))
        hxs = hx.reshape(NP, 2, DH).transpose(1, 0, 2)
        aggrp = _edge_kernel(hxs, eembs, row4d, col4d, norm, zeros_big)
        t, sums, sumsq = _g1(aggrp, hx, invbc, root_emb[l].reshape(1, D))
        if l < 3:
            hx = _g2mid(t, sums, sumsq, bn_gamma[l].reshape(1, D),
                        bn_beta[l].reshape(1, D),
                        W_lin[l + 1], b_lin[l + 1].reshape(1, D))
        else:
            h = _g2last(t, sums, sumsq, bn_gamma[l].reshape(1, D),
                        bn_beta[l].reshape(1, D))
    return h[:N]
